# Initial kernel scaffold; baseline (speedup 1.0000x reference)
import jax
import jax.numpy as jnp
from jax.experimental import pallas as pl


def kernel(hidden_states, router_w, w_gate, w_up, w_down):
    raise NotImplementedError("write your pallas kernel here")


# speedup vs baseline: 1.4693x; 1.4693x over previous
import functools

import jax
import jax.numpy as jnp
from jax import lax
from jax.experimental import pallas as pl
from jax.experimental.pallas import tpu as pltpu
from jax.experimental.pallas import tpu_sc as plsc

T = 2048
H = 768
E = 8
F = 768
KT = 2 * T
BM = 128
NB = KT // BM + E
P = NB * BM

NC, NS = 2, 16
NW = NC * NS
CHB = KT // NW
CHD = T // NW

_CH = 512
_EXACT = jax.lax.Precision.HIGHEST


def _routing_body(x_ref, rw_ref, pos1_ref, pos2_ref, w1b_ref, w2b_ref, be_ref):
    x = x_ref[...]
    rw = rw_ref[...]
    logits = lax.dot_general(x, rw, (((1,), (1,)), ((), ())),
                             preferred_element_type=jnp.float32)
    lane = lax.broadcasted_iota(jnp.int32, (T, E), 1)
    m1 = jnp.max(logits, axis=1, keepdims=True)
    i1 = jnp.min(jnp.where(logits == m1, lane, E), axis=1, keepdims=True)
    masked = jnp.where(lane == i1, -jnp.inf, logits)
    m2 = jnp.max(masked, axis=1, keepdims=True)
    i2 = jnp.min(jnp.where(masked == m2, lane, E), axis=1, keepdims=True)
    d = jnp.exp(m2 - m1)
    w1 = 1.0 / (1.0 + d)
    w2 = d / (1.0 + d)

    one1 = (lane == i1).astype(jnp.float32)
    one2 = (lane == i2).astype(jnp.float32)

    r = lax.broadcasted_iota(jnp.int32, (_CH, _CH), 0)
    c = lax.broadcasted_iota(jnp.int32, (_CH, _CH), 1)
    L = (c < r).astype(jnp.float32)
    run = jnp.zeros((1, E), jnp.float32)
    ranks = []
    for part in (one1, one2):
        for s in range(T // _CH):
            v = lax.slice(part, (s * _CH, 0), ((s + 1) * _CH, E))
            excl = lax.dot_general(L, v, (((1,), (0,)), ((), ())),
                                   preferred_element_type=jnp.float32,
                                   precision=_EXACT) + run
            run = run + jnp.sum(v, axis=0, keepdims=True)
            ranks.append(excl)
    n_parts = T // _CH
    r1 = jnp.concatenate(ranks[:n_parts], axis=0)
    r2 = jnp.concatenate(ranks[n_parts:], axis=0)

    count = run
    pcnt = jnp.floor((count + (BM - 1)) * (1.0 / BM)) * BM
    er = lax.broadcasted_iota(jnp.int32, (E, E), 0)
    ec = lax.broadcasted_iota(jnp.int32, (E, E), 1)
    U = (er < ec).astype(jnp.float32)
    poff = lax.dot_general(pcnt, U, (((1,), (0,)), ((), ())),
                           preferred_element_type=jnp.float32,
                           precision=_EXACT)
    pend = poff + pcnt

    pos1 = jnp.sum(one1 * (r1 + poff), axis=1, keepdims=True)
    pos2 = jnp.sum(one2 * (r2 + poff), axis=1, keepdims=True)
    pos1_ref[...] = pos1.astype(jnp.int32)
    pos2_ref[...] = pos2.astype(jnp.int32)

    ones128 = jnp.ones((1, 128), jnp.float32)
    w1b_ref[...] = w1 * ones128
    w2b_ref[...] = w2 * ones128

    bstart = (lax.broadcasted_iota(jnp.int32, (128, 1), 0) * BM).astype(jnp.float32)
    be = jnp.sum((pend <= bstart).astype(jnp.float32), axis=1, keepdims=True)
    be_ref[...] = jnp.minimum(be, 7.0).astype(jnp.int32)


_routing = pl.pallas_call(
    _routing_body,
    out_shape=(
        jax.ShapeDtypeStruct((T, 1), jnp.int32),
        jax.ShapeDtypeStruct((T, 1), jnp.int32),
        jax.ShapeDtypeStruct((T, 128), jnp.float32),
        jax.ShapeDtypeStruct((T, 128), jnp.float32),
        jax.ShapeDtypeStruct((128, 1), jnp.int32),
    ),
)

def _dispatch_body(x_hbm, pos_hbm, wb_hbm, xs_hbm, ws_hbm, idx_v, rows_v, wb_v,
                   sem1, sem2):
    wid = lax.axis_index("s") * NC + lax.axis_index("c")
    base = wid * CHB
    tb = lax.rem(base, T)
    pltpu.sync_copy(pos_hbm.at[pl.ds(base, CHB)], idx_v)
    pltpu.sync_copy(x_hbm.at[pl.ds(tb, CHB)], rows_v)
    pltpu.sync_copy(wb_hbm.at[pl.ds(base, CHB)], wb_v)
    cp1 = pltpu.async_copy(rows_v, xs_hbm.at[idx_v], sem1)
    cp2 = pltpu.async_copy(wb_v, ws_hbm.at[idx_v], sem2)
    cp1.wait()
    cp2.wait()


def _ffn_body(be_ref, xs_ref, wg_ref, wu_ref, wd_ref, ws_ref, ys_ref):
    xb = xs_ref[...]
    g = lax.dot_general(xb, wg_ref[0], (((1,), (1,)), ((), ())),
                        preferred_element_type=jnp.float32)
    u = lax.dot_general(xb, wu_ref[0], (((1,), (1,)), ((), ())),
                        preferred_element_type=jnp.float32)
    hcur = g * (1.0 / (1.0 + jnp.exp(-g))) * u
    yb = lax.dot_general(hcur, wd_ref[0], (((1,), (1,)), ((), ())),
                         preferred_element_type=jnp.float32)
    ys_ref[...] = yb * ws_ref[0, :, 0:1]


_ffn = pl.pallas_call(
    _ffn_body,
    grid_spec=pltpu.PrefetchScalarGridSpec(
        num_scalar_prefetch=1,
        grid=(NB,),
        in_specs=[
            pl.BlockSpec((BM, H), lambda b, be: (b, 0)),
            pl.BlockSpec((1, F, H), lambda b, be: (be[b], 0, 0)),
            pl.BlockSpec((1, F, H), lambda b, be: (be[b], 0, 0)),
            pl.BlockSpec((1, H, F), lambda b, be: (be[b], 0, 0)),
            pl.BlockSpec((1, BM, 128), lambda b, be: (b, 0, 0)),
        ],
        out_specs=pl.BlockSpec((BM, H), lambda b, be: (b, 0)),
    ),
    out_shape=jax.ShapeDtypeStruct((P, H), jnp.float32),
)


def _combine_body(ys_hbm, pos_hbm, out_hbm, idx1_v, idx2_v, buf1, buf2, sem):
    wid = lax.axis_index("s") * NC + lax.axis_index("c")
    t0 = wid * CHD
    pltpu.sync_copy(pos_hbm.at[pl.ds(t0, CHD)], idx1_v)
    pltpu.sync_copy(pos_hbm.at[pl.ds(T + t0, CHD)], idx2_v)
    pltpu.async_copy(ys_hbm.at[idx1_v], buf1, sem).wait()
    pltpu.async_copy(ys_hbm.at[idx2_v], buf2, sem).wait()

    def body(i, carry):
        for j in range(H // 16):
            sl = pl.ds(j * 16, 16)
            buf1[i, sl] = buf1[i, sl] + buf2[i, sl]
        return carry

    lax.fori_loop(0, CHD, body, 0)
    pltpu.sync_copy(buf1, out_hbm.at[pl.ds(t0, CHD)])


@functools.cache
def _sc_kernels():
    mesh = plsc.VectorSubcoreMesh(core_axis_name="c", subcore_axis_name="s",
                                  num_cores=NC, num_subcores=NS)
    dispatch = pl.kernel(
        _dispatch_body,
        out_type=(
            jax.ShapeDtypeStruct((P, H), jnp.float32),
            jax.ShapeDtypeStruct((P, 128), jnp.float32),
        ),
        mesh=mesh,
        scratch_types=[
            pltpu.VMEM((CHB,), jnp.int32),
            pltpu.VMEM((CHB, H), jnp.float32),
            pltpu.VMEM((CHB, 128), jnp.float32),
            pltpu.SemaphoreType.DMA,
            pltpu.SemaphoreType.DMA,
        ],
    )
    combine = pl.kernel(
        _combine_body,
        out_type=jax.ShapeDtypeStruct((T, H), jnp.float32),
        mesh=mesh,
        scratch_types=[
            pltpu.VMEM((CHD,), jnp.int32),
            pltpu.VMEM((CHD,), jnp.int32),
            pltpu.VMEM((CHD, H), jnp.float32),
            pltpu.VMEM((CHD, H), jnp.float32),
            pltpu.SemaphoreType.DMA,
        ],
    )
    return dispatch, combine


def kernel(hidden_states, router_w, w_gate, w_up, w_down):
    b, s, h = hidden_states.shape
    dispatch, combine = _sc_kernels()
    x = hidden_states.reshape(b * s, h)
    pos1, pos2, w1b, w2b, be = _routing(x, router_w)
    pos_flat = jnp.concatenate([pos1.reshape(T), pos2.reshape(T)], axis=0)
    wb = jnp.concatenate([w1b, w2b], axis=0)
    xs, ws = dispatch(x, pos_flat, wb)
    ys = _ffn(be.reshape(128)[:NB], xs, w_gate, w_up, w_down,
              ws.reshape(NB, BM, 128))
    out = combine(ys, pos_flat)
    return out.reshape(b, s, h)


# speedup vs baseline: 1.5721x; 1.0700x over previous
import functools

import jax
import jax.numpy as jnp
from jax import lax
from jax.experimental import pallas as pl
from jax.experimental.pallas import tpu as pltpu
from jax.experimental.pallas import tpu_sc as plsc

T = 2048
H = 768
E = 8
F = 768
KT = 2 * T
BM = 128
NB = KT // BM + E
P = NB * BM

NC, NS = 2, 16
NW = NC * NS
CHB = KT // NW
CHD = T // NW

_CH = 512
HCH = CHB // 2


def _routing_body(x_ref, rw_ref, pos1_ref, pos2_ref, w1b_ref, w2b_ref, be_ref):
    x = x_ref[...]
    rw = rw_ref[...]
    logits = lax.dot_general(x, rw, (((1,), (1,)), ((), ())),
                             preferred_element_type=jnp.float32)
    lane = lax.broadcasted_iota(jnp.int32, (T, E), 1)
    m1 = jnp.max(logits, axis=1, keepdims=True)
    i1 = jnp.min(jnp.where(logits == m1, lane, E), axis=1, keepdims=True)
    masked = jnp.where(lane == i1, -jnp.inf, logits)
    m2 = jnp.max(masked, axis=1, keepdims=True)
    i2 = jnp.min(jnp.where(masked == m2, lane, E), axis=1, keepdims=True)
    d = jnp.exp(m2 - m1)
    w1 = 1.0 / (1.0 + d)
    w2 = d / (1.0 + d)

    one1 = (lane == i1).astype(jnp.float32)
    one2 = (lane == i2).astype(jnp.float32)

    r = lax.broadcasted_iota(jnp.int32, (_CH, _CH), 0)
    c = lax.broadcasted_iota(jnp.int32, (_CH, _CH), 1)
    L = (c < r).astype(jnp.float32)
    run = jnp.zeros((1, E), jnp.float32)
    ranks = []
    for part in (one1, one2):
        for s in range(T // _CH):
            v = lax.slice(part, (s * _CH, 0), ((s + 1) * _CH, E))
            excl = lax.dot_general(L, v, (((1,), (0,)), ((), ())),
                                   preferred_element_type=jnp.float32) + run
            run = run + jnp.sum(v, axis=0, keepdims=True)
            ranks.append(excl)
    n_parts = T // _CH
    r1 = jnp.concatenate(ranks[:n_parts], axis=0)
    r2 = jnp.concatenate(ranks[n_parts:], axis=0)

    count = run
    pcnt = jnp.floor((count + (BM - 1)) * (1.0 / BM)) * BM
    er = lax.broadcasted_iota(jnp.int32, (E, E), 0)
    ec = lax.broadcasted_iota(jnp.int32, (E, E), 1)
    U = (er < ec).astype(jnp.float32)
    poff = lax.dot_general(pcnt, U, (((1,), (0,)), ((), ())),
                           preferred_element_type=jnp.float32)
    pend = poff + pcnt

    pos1 = jnp.sum(one1 * (r1 + poff), axis=1, keepdims=True)
    pos2 = jnp.sum(one2 * (r2 + poff), axis=1, keepdims=True)
    pos1_ref[...] = pos1.astype(jnp.int32)
    pos2_ref[...] = pos2.astype(jnp.int32)

    ones128 = jnp.ones((1, 128), jnp.float32)
    w1b_ref[...] = w1 * ones128
    w2b_ref[...] = w2 * ones128

    bstart = (lax.broadcasted_iota(jnp.int32, (128, 1), 0) * BM).astype(jnp.float32)
    be = jnp.sum((pend <= bstart).astype(jnp.float32), axis=1, keepdims=True)
    be_ref[...] = jnp.minimum(be, 7.0).astype(jnp.int32)


_routing = pl.pallas_call(
    _routing_body,
    out_shape=(
        jax.ShapeDtypeStruct((T, 1), jnp.int32),
        jax.ShapeDtypeStruct((T, 1), jnp.int32),
        jax.ShapeDtypeStruct((T, 128), jnp.float32),
        jax.ShapeDtypeStruct((T, 128), jnp.float32),
        jax.ShapeDtypeStruct((128, 1), jnp.int32),
    ),
)

def _dispatch_body(x_hbm, p1_hbm, p2_hbm, w1_hbm, w2_hbm, xs_hbm, ws_hbm,
                   idx_v, rows_v, wb_v, lsem0, lsem1, ssem1, ssem2):
    wid = lax.axis_index("s") * NC + lax.axis_index("c")
    tb = lax.rem(wid * CHB, T)
    k0 = wid < NS
    cx0 = pltpu.async_copy(x_hbm.at[pl.ds(tb, HCH)], rows_v.at[0], lsem0)
    cx1 = pltpu.async_copy(x_hbm.at[pl.ds(tb + HCH, HCH)], rows_v.at[1], lsem1)

    @pl.when(k0)
    def _():
        for h in range(2):
            off = tb + h * HCH
            pltpu.sync_copy(p1_hbm.at[pl.ds(off, HCH)], idx_v.at[h])
            pltpu.sync_copy(w1_hbm.at[pl.ds(off, HCH)], wb_v.at[h])

    @pl.when(jnp.logical_not(k0))
    def _():
        for h in range(2):
            off = tb + h * HCH
            pltpu.sync_copy(p2_hbm.at[pl.ds(off, HCH)], idx_v.at[h])
            pltpu.sync_copy(w2_hbm.at[pl.ds(off, HCH)], wb_v.at[h])

    cx0.wait()
    s0a = pltpu.async_copy(rows_v.at[0], xs_hbm.at[idx_v.at[0]], ssem1)
    s0b = pltpu.async_copy(wb_v.at[0], ws_hbm.at[idx_v.at[0]], ssem2)
    cx1.wait()
    s1a = pltpu.async_copy(rows_v.at[1], xs_hbm.at[idx_v.at[1]], ssem1)
    s1b = pltpu.async_copy(wb_v.at[1], ws_hbm.at[idx_v.at[1]], ssem2)
    s0a.wait()
    s0b.wait()
    s1a.wait()
    s1b.wait()


def _ffn_body(be_ref, xs_ref, wg_ref, wu_ref, wd_ref, ws_ref, ys_ref):
    xb = xs_ref[...]
    g = lax.dot_general(xb, wg_ref[0], (((1,), (1,)), ((), ())),
                        preferred_element_type=jnp.float32)
    u = lax.dot_general(xb, wu_ref[0], (((1,), (1,)), ((), ())),
                        preferred_element_type=jnp.float32)
    hcur = g * (1.0 / (1.0 + jnp.exp(-g))) * u
    yb = lax.dot_general(hcur, wd_ref[0], (((1,), (1,)), ((), ())),
                         preferred_element_type=jnp.float32)
    ys_ref[...] = yb * ws_ref[0, :, 0:1]


_ffn = pl.pallas_call(
    _ffn_body,
    grid_spec=pltpu.PrefetchScalarGridSpec(
        num_scalar_prefetch=1,
        grid=(NB,),
        in_specs=[
            pl.BlockSpec((BM, H), lambda b, be: (b, 0)),
            pl.BlockSpec((1, F, H), lambda b, be: (be[b], 0, 0)),
            pl.BlockSpec((1, F, H), lambda b, be: (be[b], 0, 0)),
            pl.BlockSpec((1, H, F), lambda b, be: (be[b], 0, 0)),
            pl.BlockSpec((1, BM, 128), lambda b, be: (b, 0, 0)),
        ],
        out_specs=pl.BlockSpec((BM, H), lambda b, be: (b, 0)),
    ),
    out_shape=jax.ShapeDtypeStruct((P, H), jnp.float32),
)


def _combine_body(ys_hbm, p1_hbm, p2_hbm, out_hbm, idx1_v, idx2_v, buf1, buf2,
                  sem1, sem2):
    wid = lax.axis_index("s") * NC + lax.axis_index("c")
    t0 = wid * CHD
    pltpu.sync_copy(p1_hbm.at[pl.ds(t0, CHD)], idx1_v)
    pltpu.sync_copy(p2_hbm.at[pl.ds(t0, CHD)], idx2_v)
    c1 = pltpu.async_copy(ys_hbm.at[idx1_v], buf1, sem1)
    c2 = pltpu.async_copy(ys_hbm.at[idx2_v], buf2, sem2)
    c1.wait()
    c2.wait()

    def body(i, carry):
        for j in range(H // 16):
            sl = pl.ds(j * 16, 16)
            buf1[i, sl] = buf1[i, sl] + buf2[i, sl]
        return carry

    lax.fori_loop(0, CHD, body, 0)
    pltpu.sync_copy(buf1, out_hbm.at[pl.ds(t0, CHD)])


@functools.cache
def _sc_kernels():
    mesh = plsc.VectorSubcoreMesh(core_axis_name="c", subcore_axis_name="s",
                                  num_cores=NC, num_subcores=NS)
    dispatch = pl.kernel(
        _dispatch_body,
        out_type=(
            jax.ShapeDtypeStruct((P, H), jnp.float32),
            jax.ShapeDtypeStruct((P, 128), jnp.float32),
        ),
        mesh=mesh,
        scratch_types=[
            pltpu.VMEM((2, HCH), jnp.int32),
            pltpu.VMEM((2, HCH, H), jnp.float32),
            pltpu.VMEM((2, HCH, 128), jnp.float32),
            pltpu.SemaphoreType.DMA,
            pltpu.SemaphoreType.DMA,
            pltpu.SemaphoreType.DMA,
            pltpu.SemaphoreType.DMA,
        ],
    )
    combine = pl.kernel(
        _combine_body,
        out_type=jax.ShapeDtypeStruct((T, H), jnp.float32),
        mesh=mesh,
        scratch_types=[
            pltpu.VMEM((CHD,), jnp.int32),
            pltpu.VMEM((CHD,), jnp.int32),
            pltpu.VMEM((CHD, H), jnp.float32),
            pltpu.VMEM((CHD, H), jnp.float32),
            pltpu.SemaphoreType.DMA,
            pltpu.SemaphoreType.DMA,
        ],
    )
    return dispatch, combine


def kernel(hidden_states, router_w, w_gate, w_up, w_down):
    b, s, h = hidden_states.shape
    dispatch, combine = _sc_kernels()
    x = hidden_states.reshape(b * s, h)
    pos1, pos2, w1b, w2b, be = _routing(x, router_w)
    p1 = pos1.reshape(T)
    p2 = pos2.reshape(T)
    xs, ws = dispatch(x, p1, p2, w1b, w2b)
    ys = _ffn(be.reshape(128)[:NB], xs, w_gate, w_up, w_down,
              ws.reshape(NB, BM, 128))
    out = combine(ys, p1, p2)
    return out.reshape(b, s, h)


# speedup vs baseline: 2.1159x; 1.3459x over previous
import functools

import jax
import jax.numpy as jnp
from jax import lax
from jax.experimental import pallas as pl
from jax.experimental.pallas import tpu as pltpu
from jax.experimental.pallas import tpu_sc as plsc

T = 2048
H = 768
E = 8
F = 768
KT = 2 * T
BM = 512
NB = 15
P = NB * BM

NC, NS = 2, 16
NW = NC * NS
CHB = KT // NW
CHD = T // NW

_CH = 512


def _routing_body(x_ref, rw_ref, pos1_ref, pos2_ref, w1b_ref, w2b_ref, be_ref):
    x = x_ref[...]
    rw = rw_ref[...]
    logits = lax.dot_general(x, rw, (((1,), (1,)), ((), ())),
                             preferred_element_type=jnp.float32)
    lane = lax.broadcasted_iota(jnp.int32, (T, E), 1)
    m1 = jnp.max(logits, axis=1, keepdims=True)
    i1 = jnp.min(jnp.where(logits == m1, lane, E), axis=1, keepdims=True)
    masked = jnp.where(lane == i1, -jnp.inf, logits)
    m2 = jnp.max(masked, axis=1, keepdims=True)
    i2 = jnp.min(jnp.where(masked == m2, lane, E), axis=1, keepdims=True)
    d = jnp.exp(m2 - m1)
    w1 = 1.0 / (1.0 + d)
    w2 = d / (1.0 + d)

    one1 = (lane == i1).astype(jnp.float32)
    one2 = (lane == i2).astype(jnp.float32)

    r = lax.broadcasted_iota(jnp.int32, (_CH, _CH), 0)
    c = lax.broadcasted_iota(jnp.int32, (_CH, _CH), 1)
    L = (c < r).astype(jnp.float32)
    run = jnp.zeros((1, E), jnp.float32)
    ranks = []
    for part in (one1, one2):
        for s in range(T // _CH):
            v = lax.slice(part, (s * _CH, 0), ((s + 1) * _CH, E))
            excl = lax.dot_general(L, v, (((1,), (0,)), ((), ())),
                                   preferred_element_type=jnp.float32) + run
            run = run + jnp.sum(v, axis=0, keepdims=True)
            ranks.append(excl)
    n_parts = T // _CH
    r1 = jnp.concatenate(ranks[:n_parts], axis=0)
    r2 = jnp.concatenate(ranks[n_parts:], axis=0)

    count = run
    pcnt = jnp.floor((count + (BM - 1)) * (1.0 / BM)) * BM
    er = lax.broadcasted_iota(jnp.int32, (E, E), 0)
    ec = lax.broadcasted_iota(jnp.int32, (E, E), 1)
    U = (er < ec).astype(jnp.float32)
    poff = lax.dot_general(pcnt, U, (((1,), (0,)), ((), ())),
                           preferred_element_type=jnp.float32)
    pend = poff + pcnt

    pos1 = jnp.sum(one1 * (r1 + poff), axis=1, keepdims=True)
    pos2 = jnp.sum(one2 * (r2 + poff), axis=1, keepdims=True)
    pos1_ref[...] = pos1.astype(jnp.int32)
    pos2_ref[...] = pos2.astype(jnp.int32)

    ones16 = jnp.ones((1, 16), jnp.float32)
    w1b_ref[...] = w1 * ones16
    w2b_ref[...] = w2 * ones16

    nlive = jnp.sum(pcnt, axis=1, keepdims=True) * (1.0 / BM)
    ri = lax.broadcasted_iota(jnp.int32, (128, 1), 0)
    bstart = (ri * BM).astype(jnp.float32)
    be = jnp.sum((pend <= bstart).astype(jnp.float32), axis=1, keepdims=True)
    be = jnp.minimum(be, 7.0)
    be_ref[...] = jnp.where(ri == 64, nlive, be).astype(jnp.int32)


_routing = pl.pallas_call(
    _routing_body,
    out_shape=(
        jax.ShapeDtypeStruct((T, 1), jnp.int32),
        jax.ShapeDtypeStruct((T, 1), jnp.int32),
        jax.ShapeDtypeStruct((T, 16), jnp.float32),
        jax.ShapeDtypeStruct((T, 16), jnp.float32),
        jax.ShapeDtypeStruct((128, 1), jnp.int32),
    ),
)

NQ = 4
QCH = CHB // NQ


def _dispatch_body(x_hbm, p1_hbm, p2_hbm, xs_hbm, idx_v, rows_v,
                   lsem0, lsem1, lsem2, lsem3, ssem):
    wid = lax.axis_index("s") * NC + lax.axis_index("c")
    tb = lax.rem(wid * CHB, T)
    k0 = wid < NS
    lsems = (lsem0, lsem1, lsem2, lsem3)
    loads = [pltpu.async_copy(x_hbm.at[pl.ds(tb + q * QCH, QCH)],
                              rows_v.at[q], lsems[q]) for q in range(NQ)]

    @pl.when(k0)
    def _():
        for q in range(NQ):
            pltpu.sync_copy(p1_hbm.at[pl.ds(tb + q * QCH, QCH)], idx_v.at[q])

    @pl.when(jnp.logical_not(k0))
    def _():
        for q in range(NQ):
            pltpu.sync_copy(p2_hbm.at[pl.ds(tb + q * QCH, QCH)], idx_v.at[q])

    stores = []
    for q in range(NQ):
        loads[q].wait()
        stores.append(pltpu.async_copy(rows_v.at[q], xs_hbm.at[idx_v.at[q]],
                                       ssem))
    for s in stores:
        s.wait()


def _ffn_body(be_ref, xs_ref, wg_ref, wu_ref, wd_ref, ys_ref):
    b = pl.program_id(0)

    @pl.when(b < be_ref[64])
    def _():
        xb = xs_ref[...]
        g = lax.dot_general(xb, wg_ref[0], (((1,), (1,)), ((), ())),
                            preferred_element_type=jnp.float32)
        u = lax.dot_general(xb, wu_ref[0], (((1,), (1,)), ((), ())),
                            preferred_element_type=jnp.float32)
        hcur = g * (1.0 / (1.0 + jnp.exp(-g))) * u
        ys_ref[...] = lax.dot_general(hcur, wd_ref[0],
                                      (((1,), (1,)), ((), ())),
                                      preferred_element_type=jnp.float32)


def _blive(b, be):
    return jnp.minimum(b, be[64] - 1)


_ffn = pl.pallas_call(
    _ffn_body,
    grid_spec=pltpu.PrefetchScalarGridSpec(
        num_scalar_prefetch=1,
        grid=(NB,),
        in_specs=[
            pl.BlockSpec((BM, H), lambda b, be: (_blive(b, be), 0)),
            pl.BlockSpec((1, F, H), lambda b, be: (be[_blive(b, be)], 0, 0)),
            pl.BlockSpec((1, F, H), lambda b, be: (be[_blive(b, be)], 0, 0)),
            pl.BlockSpec((1, H, F), lambda b, be: (be[_blive(b, be)], 0, 0)),
        ],
        out_specs=pl.BlockSpec((BM, H), lambda b, be: (_blive(b, be), 0)),
    ),
    out_shape=jax.ShapeDtypeStruct((P, H), jnp.float32),
)


HD = CHD // 2


def _combine_body(ys_hbm, p1_hbm, p2_hbm, w1_hbm, w2_hbm, out_hbm,
                  idx1_v, idx2_v, w1_v, w2_v, buf1, buf2, sem1, sem2, osem):
    wid = lax.axis_index("s") * NC + lax.axis_index("c")
    t0 = wid * CHD
    pltpu.sync_copy(p1_hbm.at[pl.ds(t0, HD)], idx1_v.at[0])
    pltpu.sync_copy(p2_hbm.at[pl.ds(t0, HD)], idx2_v.at[0])
    c1a = pltpu.async_copy(ys_hbm.at[idx1_v.at[0]], buf1.at[0], sem1)
    c2a = pltpu.async_copy(ys_hbm.at[idx2_v.at[0]], buf2.at[0], sem2)
    pltpu.sync_copy(p1_hbm.at[pl.ds(t0 + HD, HD)], idx1_v.at[1])
    pltpu.sync_copy(p2_hbm.at[pl.ds(t0 + HD, HD)], idx2_v.at[1])
    c1b = pltpu.async_copy(ys_hbm.at[idx1_v.at[1]], buf1.at[1], sem1)
    c2b = pltpu.async_copy(ys_hbm.at[idx2_v.at[1]], buf2.at[1], sem2)
    pltpu.sync_copy(w1_hbm.at[pl.ds(t0, CHD)], w1_v)
    pltpu.sync_copy(w2_hbm.at[pl.ds(t0, CHD)], w2_v)

    def half(h, c1, c2):
        c1.wait()
        c2.wait()

        def body(i, carry):
            wv1 = w1_v[h * HD + i]
            wv2 = w2_v[h * HD + i]
            for jj in range(H // 16):
                sl = pl.ds(jj * 16, 16)
                buf1[h, i, sl] = buf1[h, i, sl] * wv1 + buf2[h, i, sl] * wv2
            return carry

        lax.fori_loop(0, HD, body, 0)
        return pltpu.async_copy(buf1.at[h], out_hbm.at[pl.ds(t0 + h * HD, HD)],
                                osem)

    o0 = half(0, c1a, c2a)
    o1 = half(1, c1b, c2b)
    o0.wait()
    o1.wait()


@functools.cache
def _sc_kernels():
    mesh = plsc.VectorSubcoreMesh(core_axis_name="c", subcore_axis_name="s",
                                  num_cores=NC, num_subcores=NS)
    dispatch = pl.kernel(
        _dispatch_body,
        out_type=jax.ShapeDtypeStruct((P, H), jnp.float32),
        mesh=mesh,
        scratch_types=[
            pltpu.VMEM((NQ, QCH), jnp.int32),
            pltpu.VMEM((NQ, QCH, H), jnp.float32),
            pltpu.SemaphoreType.DMA,
            pltpu.SemaphoreType.DMA,
            pltpu.SemaphoreType.DMA,
            pltpu.SemaphoreType.DMA,
            pltpu.SemaphoreType.DMA,
        ],
    )
    combine = pl.kernel(
        _combine_body,
        out_type=jax.ShapeDtypeStruct((T, H), jnp.float32),
        mesh=mesh,
        scratch_types=[
            pltpu.VMEM((2, HD), jnp.int32),
            pltpu.VMEM((2, HD), jnp.int32),
            pltpu.VMEM((CHD, 16), jnp.float32),
            pltpu.VMEM((CHD, 16), jnp.float32),
            pltpu.VMEM((2, HD, H), jnp.float32),
            pltpu.VMEM((2, HD, H), jnp.float32),
            pltpu.SemaphoreType.DMA,
            pltpu.SemaphoreType.DMA,
            pltpu.SemaphoreType.DMA,
        ],
    )
    return dispatch, combine


def kernel(hidden_states, router_w, w_gate, w_up, w_down):
    b, s, h = hidden_states.shape
    dispatch, combine = _sc_kernels()
    x = hidden_states.reshape(b * s, h)
    pos1, pos2, w1b, w2b, be = _routing(x, router_w)
    p1 = pos1.reshape(T)
    p2 = pos2.reshape(T)
    xs = dispatch(x, p1, p2)
    ys = _ffn(be.reshape(128), xs, w_gate, w_up, w_down)
    out = combine(ys, p1, p2, w1b, w2b)
    return out.reshape(b, s, h)
